# baseline (speedup 1.0000x reference)
import functools

import jax
import jax.numpy as jnp
from jax import lax
from jax.experimental import pallas as pl
from jax.experimental.pallas import tpu as pltpu
from jax.experimental.pallas import tpu_sc as plsc

N_NODES = 100000
N_EDGES = 6400000

_info = plsc.get_sparse_core_info()
NC, NS, NL = _info.num_cores, _info.num_subcores, _info.num_lanes
NW = NC * NS

CHUNK = 5120
NBLK = CHUNK // 128
NCHUNKS = N_EDGES // CHUNK
ITERS = -(-NCHUNKS // NW)
NSTEPS = 2 * ITERS
OUT_WORDS = 4 * N_EDGES


def _make_kernel():
    mesh = plsc.VectorSubcoreMesh(core_axis_name="c", subcore_axis_name="s")

    @functools.partial(
        pl.kernel,
        mesh=mesh,
        out_type=[
            jax.ShapeDtypeStruct((OUT_WORDS,), jnp.float32),
            jax.ShapeDtypeStruct((OUT_WORDS,), jnp.float32),
        ],
        scratch_types=[
            pltpu.VMEM((2, 2, CHUNK), jnp.int32),
            pltpu.VMEM((4 * CHUNK,), jnp.int32),
            pltpu.VMEM((4 * CHUNK,), jnp.int32),
            pltpu.VMEM((4 * CHUNK,), jnp.float32),
            pltpu.VMEM((4 * CHUNK,), jnp.float32),
            pltpu.VMEM_SHARED((3 * N_NODES,), jnp.float32),
            pltpu.SemaphoreType.DMA,
            pltpu.SemaphoreType.DMA,
            pltpu.SemaphoreType.DMA,
        ],
    )
    def k(pos_hbm, edge_hbm, out0_hbm, out1_hbm,
          idx_v, exp0_v, exp1_v, vals0_v, vals1_v, pos_sp,
          sem_g, sem_o, sem_i):
        cid = lax.axis_index("c")
        sid = lax.axis_index("s")
        wid = sid * NC + cid
        iota = lax.iota(jnp.int32, NL)
        exps = (exp0_v, exp1_v)
        valss = (vals0_v, vals1_v)
        outs = (out0_hbm, out1_hbm)

        @pl.when(sid == 0)
        def _():
            pltpu.sync_copy(pos_hbm, pos_sp)

        plsc.subcore_barrier()

        def fill_pads(b, c2):
            for mm in range(8):
                v = (b * 128 + NL * mm + iota) & 262143
                for e in exps:
                    e[pl.ds(512 * b + 384 + NL * mm, NL)] = v
            return c2

        lax.fori_loop(0, NBLK, fill_pads, 0)


        def chunk_of(s):
            return (s // 2) * NW + wid

        def valid(s):
            return chunk_of(s) < NCHUNKS

        def prepare(s, par, q):
            @pl.when(valid(s))
            def _():
                def blk(b, c2):
                    for mm in range(8):
                        tv = idx_v[q, par, pl.ds(128 * b + NL * mm, NL)] * 3
                        o = 512 * b + NL * mm
                        exps[par][pl.ds(o, NL)] = tv
                        exps[par][pl.ds(o + 128, NL)] = tv + 1
                        exps[par][pl.ds(o + 256, NL)] = tv + 2
                    return c2

                lax.fori_loop(0, NBLK, blk, 0)

        def issue_idx(i):
            @pl.when((i * NW + wid) < NCHUNKS)
            def _():
                pltpu.async_copy(
                    edge_hbm.at[:, pl.ds((i * NW + wid) * CHUNK, CHUNK)],
                    idx_v.at[i % 2], sem_i)

        def wait_idx(i):
            @pl.when((i * NW + wid) < NCHUNKS)
            def _():
                pltpu.make_async_copy(
                    edge_hbm.at[:, pl.ds(0, CHUNK)],
                    idx_v.at[i % 2], sem_i).wait()

        def issue_gather(s, par):
            @pl.when(valid(s))
            def _():
                pltpu.async_copy(pos_sp.at[exps[par]], valss[par], sem_g)

        def wait_gather(s, par):
            @pl.when(valid(s))
            def _():
                pltpu.make_async_copy(
                    pos_sp.at[exps[par]], valss[par], sem_g).wait()

        def issue_out(s, par):
            @pl.when(valid(s))
            def _():
                pltpu.async_copy(
                    valss[par],
                    outs[par].at[pl.ds(4 * chunk_of(s) * CHUNK, 4 * CHUNK)],
                    sem_o)

        def wait_out(s, extra=None):
            pred = valid(s) if extra is None else (valid(s) & extra)

            @pl.when(pred)
            def _():
                pltpu.make_async_copy(
                    vals0_v, out0_hbm.at[pl.ds(0, 4 * CHUNK)], sem_o).wait()

        issue_idx(0)
        wait_idx(0)
        issue_idx(1)
        prepare(0, 0, 0)
        issue_gather(0, 0)

        def body(t, carry):
            for par in (0, 1):
                s = 2 * t + par
                par1 = 1 - par
                if par == 1:
                    wait_idx(t + 1)
                    issue_idx(t + 2)
                    q1 = (t + 1) % 2
                else:
                    q1 = t % 2
                prepare(s + 1, par1, q1)
                wait_gather(s, par)
                wait_out(s - 1, s >= 1)
                issue_out(s, par)
                issue_gather(s + 1, par1)
            return carry

        lax.fori_loop(0, ITERS, body, 0)

        wait_out(NSTEPS - 1)

    return k


_kernel = _make_kernel()


def kernel(position, edge_index):
    out0, out1 = _kernel(position.reshape(-1), edge_index)

    def as2d(flat):
        y = flat.reshape(N_EDGES // 128, 4, 128).transpose(0, 2, 1)
        return y.reshape(N_EDGES, 4)[:, :3]

    return (as2d(out0), as2d(out1))


# speedup vs baseline: 1.2116x; 1.2116x over previous
import functools

import jax
import jax.numpy as jnp
from jax import lax
from jax.experimental import pallas as pl
from jax.experimental.pallas import tpu as pltpu
from jax.experimental.pallas import tpu_sc as plsc

N_NODES = 100000
N_EDGES = 6400000

_info = plsc.get_sparse_core_info()
NC, NS, NL = _info.num_cores, _info.num_subcores, _info.num_lanes
NW = NC * NS

CHUNK = 5120
NBLK = CHUNK // 128
NCHUNKS = N_EDGES // CHUNK
ITERS = -(-NCHUNKS // NW)
NSTEPS = 2 * ITERS
OUT_WORDS = 4 * N_EDGES


def _make_kernel():
    mesh = plsc.VectorSubcoreMesh(core_axis_name="c", subcore_axis_name="s")

    @functools.partial(
        pl.kernel,
        mesh=mesh,
        out_type=[
            jax.ShapeDtypeStruct((OUT_WORDS,), jnp.float32),
            jax.ShapeDtypeStruct((OUT_WORDS,), jnp.float32),
        ],
        scratch_types=[
            pltpu.VMEM((2, 2, CHUNK), jnp.int32),
            pltpu.VMEM((3 * CHUNK,), jnp.int32),
            pltpu.VMEM((3 * CHUNK,), jnp.int32),
            pltpu.VMEM((3 * CHUNK,), jnp.float32),
            pltpu.VMEM((3 * CHUNK,), jnp.float32),
            pltpu.VMEM_SHARED((3 * N_NODES,), jnp.float32),
            pltpu.SemaphoreType.DMA,
            pltpu.SemaphoreType.DMA,
            pltpu.SemaphoreType.DMA,
        ],
    )
    def k(pos_hbm, edge_hbm, out0_hbm, out1_hbm,
          idx_v, exp0_v, exp1_v, vals0_v, vals1_v, pos_sp,
          sem_g, sem_o, sem_i):
        cid = lax.axis_index("c")
        sid = lax.axis_index("s")
        wid = sid * NC + cid
        iota = lax.iota(jnp.int32, NL)
        exps = (exp0_v, exp1_v)
        valss = (vals0_v, vals1_v)
        outs = (out0_hbm, out1_hbm)

        @pl.when(sid == 0)
        def _():
            pltpu.sync_copy(pos_hbm, pos_sp)

        plsc.subcore_barrier()


        def chunk_of(s):
            return (s // 2) * NW + wid

        def valid(s):
            return chunk_of(s) < NCHUNKS

        def prepare(s, par, q):
            @pl.when(valid(s))
            def _():
                def blk(b, c2):
                    for mm in range(8):
                        tv = idx_v[q, par, pl.ds(128 * b + NL * mm, NL)] * 3
                        o = 384 * b + NL * mm
                        exps[par][pl.ds(o, NL)] = tv
                        exps[par][pl.ds(o + 128, NL)] = tv + 1
                        exps[par][pl.ds(o + 256, NL)] = tv + 2
                    return c2

                lax.fori_loop(0, NBLK, blk, 0)

        def issue_idx(i):
            @pl.when((i * NW + wid) < NCHUNKS)
            def _():
                pltpu.async_copy(
                    edge_hbm.at[:, pl.ds((i * NW + wid) * CHUNK, CHUNK)],
                    idx_v.at[i % 2], sem_i)

        def wait_idx(i):
            @pl.when((i * NW + wid) < NCHUNKS)
            def _():
                pltpu.make_async_copy(
                    edge_hbm.at[:, pl.ds(0, CHUNK)],
                    idx_v.at[i % 2], sem_i).wait()

        def issue_gather(s, par):
            @pl.when(valid(s))
            def _():
                for b in range(NBLK):
                    sl = pl.ds(384 * b, 384)
                    pltpu.async_copy(
                        pos_sp.at[exps[par].at[sl]], valss[par].at[sl], sem_g)

        def wait_gather(s, par):
            @pl.when(valid(s))
            def _():
                pltpu.make_async_copy(
                    pos_sp.at[exps[par]], valss[par], sem_g).wait()

        def issue_out(s, par):
            @pl.when(valid(s))
            def _():
                base = 4 * chunk_of(s) * CHUNK
                for b in range(NBLK):
                    pltpu.async_copy(
                        valss[par].at[pl.ds(384 * b, 384)],
                        outs[par].at[pl.ds(base + 512 * b, 384)],
                        sem_o)

        def wait_out(s, extra=None):
            pred = valid(s) if extra is None else (valid(s) & extra)

            @pl.when(pred)
            def _():
                pltpu.make_async_copy(
                    vals0_v, out0_hbm.at[pl.ds(0, 3 * CHUNK)], sem_o).wait()

        issue_idx(0)
        wait_idx(0)
        issue_idx(1)
        prepare(0, 0, 0)
        issue_gather(0, 0)

        def body(t, carry):
            for par in (0, 1):
                s = 2 * t + par
                par1 = 1 - par
                if par == 1:
                    wait_idx(t + 1)
                    issue_idx(t + 2)
                    q1 = (t + 1) % 2
                else:
                    q1 = t % 2
                prepare(s + 1, par1, q1)
                wait_gather(s, par)
                wait_out(s - 1, s >= 1)
                issue_out(s, par)
                issue_gather(s + 1, par1)
            return carry

        lax.fori_loop(0, ITERS, body, 0)

        wait_out(NSTEPS - 1)

    return k


_kernel = _make_kernel()


def kernel(position, edge_index):
    out0, out1 = _kernel(position.reshape(-1), edge_index)

    def as2d(flat):
        y = flat.reshape(N_EDGES // 128, 4, 128).transpose(0, 2, 1)
        return y.reshape(N_EDGES, 4)[:, :3]

    return (as2d(out0), as2d(out1))


# speedup vs baseline: 1.2605x; 1.0403x over previous
import functools

import jax
import jax.numpy as jnp
from jax import lax
from jax.experimental import pallas as pl
from jax.experimental.pallas import tpu as pltpu
from jax.experimental.pallas import tpu_sc as plsc

N_NODES = 100000
N_EDGES = 6400000

_info = plsc.get_sparse_core_info()
NC, NS, NL = _info.num_cores, _info.num_subcores, _info.num_lanes
NW = NC * NS

CHUNK = 5120
NBLK = CHUNK // 128
NCHUNKS = N_EDGES // CHUNK
ITERS = -(-NCHUNKS // NW)
NSTEPS = 2 * ITERS
OUT_WORDS = 4 * N_EDGES


def _make_kernel():
    mesh = plsc.VectorSubcoreMesh(core_axis_name="c", subcore_axis_name="s")

    @functools.partial(
        pl.kernel,
        mesh=mesh,
        out_type=[
            jax.ShapeDtypeStruct((OUT_WORDS,), jnp.float32),
            jax.ShapeDtypeStruct((OUT_WORDS,), jnp.float32),
        ],
        scratch_types=[
            pltpu.VMEM((2, 2, CHUNK), jnp.int32),
            pltpu.VMEM((3 * CHUNK,), jnp.int32),
            pltpu.VMEM((3 * CHUNK,), jnp.int32),
            pltpu.VMEM((4 * CHUNK,), jnp.float32),
            pltpu.VMEM((4 * CHUNK,), jnp.float32),
            pltpu.VMEM_SHARED((3 * N_NODES,), jnp.float32),
            pltpu.SemaphoreType.DMA,
            pltpu.SemaphoreType.DMA,
            pltpu.SemaphoreType.DMA,
        ],
    )
    def k(pos_hbm, edge_hbm, out0_hbm, out1_hbm,
          idx_v, exp0_v, exp1_v, vals0_v, vals1_v, pos_sp,
          sem_g, sem_o, sem_i):
        cid = lax.axis_index("c")
        sid = lax.axis_index("s")
        wid = sid * NC + cid
        iota = lax.iota(jnp.int32, NL)
        exps = (exp0_v, exp1_v)
        valss = (vals0_v, vals1_v)
        outs = (out0_hbm, out1_hbm)

        @pl.when(sid == 0)
        def _():
            pltpu.sync_copy(pos_hbm, pos_sp)

        plsc.subcore_barrier()


        def chunk_of(s):
            return (s // 2) * NW + wid

        def valid(s):
            return chunk_of(s) < NCHUNKS

        def prepare(s, par, q):
            @pl.when(valid(s))
            def _():
                def blk(b, c2):
                    for mm in range(8):
                        tv = idx_v[q, par, pl.ds(128 * b + NL * mm, NL)] * 3
                        o = 384 * b + NL * mm
                        exps[par][pl.ds(o, NL)] = tv
                        exps[par][pl.ds(o + 128, NL)] = tv + 1
                        exps[par][pl.ds(o + 256, NL)] = tv + 2
                    return c2

                lax.fori_loop(0, NBLK, blk, 0)

        def issue_idx(i):
            @pl.when((i * NW + wid) < NCHUNKS)
            def _():
                pltpu.async_copy(
                    edge_hbm.at[:, pl.ds((i * NW + wid) * CHUNK, CHUNK)],
                    idx_v.at[i % 2], sem_i)

        def wait_idx(i):
            @pl.when((i * NW + wid) < NCHUNKS)
            def _():
                pltpu.make_async_copy(
                    edge_hbm.at[:, pl.ds(0, CHUNK)],
                    idx_v.at[i % 2], sem_i).wait()

        def issue_gather(s, par):
            @pl.when(valid(s))
            def _():
                for b in range(NBLK):
                    pltpu.async_copy(
                        pos_sp.at[exps[par].at[pl.ds(384 * b, 384)]],
                        valss[par].at[pl.ds(512 * b, 384)], sem_g)

        def wait_gather(s, par):
            @pl.when(valid(s))
            def _():
                pltpu.make_async_copy(
                    pos_sp.at[exps[par]],
                    valss[par].at[pl.ds(0, 3 * CHUNK)], sem_g).wait()

        def issue_out(s, par):
            @pl.when(valid(s))
            def _():
                pltpu.async_copy(
                    valss[par],
                    outs[par].at[pl.ds(4 * chunk_of(s) * CHUNK, 4 * CHUNK)],
                    sem_o)

        def wait_out(s, extra=None):
            pred = valid(s) if extra is None else (valid(s) & extra)

            @pl.when(pred)
            def _():
                pltpu.make_async_copy(
                    vals0_v, out0_hbm.at[pl.ds(0, 4 * CHUNK)], sem_o).wait()

        issue_idx(0)
        wait_idx(0)
        issue_idx(1)
        prepare(0, 0, 0)
        issue_gather(0, 0)

        def body(t, carry):
            for par in (0, 1):
                s = 2 * t + par
                par1 = 1 - par
                if par == 1:
                    wait_idx(t + 1)
                    issue_idx(t + 2)
                    q1 = (t + 1) % 2
                else:
                    q1 = t % 2
                prepare(s + 1, par1, q1)
                wait_gather(s, par)
                wait_out(s - 1, s >= 1)
                issue_out(s, par)
                issue_gather(s + 1, par1)
            return carry

        lax.fori_loop(0, ITERS, body, 0)

        wait_out(NSTEPS - 1)

    return k


_kernel = _make_kernel()


def kernel(position, edge_index):
    out0, out1 = _kernel(position.reshape(-1), edge_index)

    def as2d(flat):
        y = flat.reshape(N_EDGES // 128, 4, 128).transpose(0, 2, 1)
        return y.reshape(N_EDGES, 4)[:, :3]

    return (as2d(out0), as2d(out1))


# speedup vs baseline: 1.2849x; 1.0194x over previous
import functools

import jax
import jax.numpy as jnp
from jax import lax
from jax.experimental import pallas as pl
from jax.experimental.pallas import tpu as pltpu
from jax.experimental.pallas import tpu_sc as plsc

N_NODES = 100000
N_EDGES = 6400000

_info = plsc.get_sparse_core_info()
NC, NS, NL = _info.num_cores, _info.num_subcores, _info.num_lanes
NW = NC * NS

CHUNK = 5120
NBLK = CHUNK // 128
NCHUNKS = N_EDGES // CHUNK
ITERS = -(-NCHUNKS // NW)
NSTEPS = 2 * ITERS
OUT_WORDS = 4 * N_EDGES


def _make_kernel():
    mesh = plsc.VectorSubcoreMesh(core_axis_name="c", subcore_axis_name="s")

    @functools.partial(
        pl.kernel,
        mesh=mesh,
        out_type=[
            jax.ShapeDtypeStruct((OUT_WORDS,), jnp.float32),
            jax.ShapeDtypeStruct((OUT_WORDS,), jnp.float32),
        ],
        scratch_types=[
            pltpu.VMEM((2, 2, CHUNK), jnp.int32),
            pltpu.VMEM((3 * CHUNK,), jnp.int32),
            pltpu.VMEM((3 * CHUNK,), jnp.int32),
            pltpu.VMEM((4 * CHUNK,), jnp.float32),
            pltpu.VMEM((4 * CHUNK,), jnp.float32),
            pltpu.VMEM_SHARED((3 * N_NODES,), jnp.float32),
            pltpu.SemaphoreType.DMA,
            pltpu.SemaphoreType.DMA,
            pltpu.SemaphoreType.DMA,
        ],
    )
    def k(pos_hbm, edge_hbm, out0_hbm, out1_hbm,
          idx_v, exp0_v, exp1_v, vals0_v, vals1_v, pos_sp,
          sem_g, sem_o, sem_i):
        cid = lax.axis_index("c")
        sid = lax.axis_index("s")
        wid = sid * NC + cid
        iota = lax.iota(jnp.int32, NL)
        exps = (exp0_v, exp1_v)
        valss = (vals0_v, vals1_v)
        outs = (out0_hbm, out1_hbm)

        @pl.when(sid == 0)
        def _():
            pltpu.sync_copy(pos_hbm, pos_sp)

        plsc.subcore_barrier()


        def chunk_of(s):
            return (s // 2) * NW + wid

        def valid(s):
            return chunk_of(s) < NCHUNKS

        def prepare(s, par, q):
            @pl.when(valid(s))
            def _():
                def blk(b, c2):
                    for mm in range(8):
                        tv = idx_v[q, par, pl.ds(128 * b + NL * mm, NL)] * 3
                        o = 384 * b + NL * mm
                        exps[par][pl.ds(o, NL)] = tv
                        exps[par][pl.ds(o + 128, NL)] = tv + 1
                        exps[par][pl.ds(o + 256, NL)] = tv + 2
                    return c2

                lax.fori_loop(0, NBLK, blk, 0)

        def issue_idx(i):
            @pl.when((i * NW + wid) < NCHUNKS)
            def _():
                pltpu.async_copy(
                    edge_hbm.at[:, pl.ds((i * NW + wid) * CHUNK, CHUNK)],
                    idx_v.at[i % 2], sem_i)

        def wait_idx(i):
            @pl.when((i * NW + wid) < NCHUNKS)
            def _():
                pltpu.make_async_copy(
                    edge_hbm.at[:, pl.ds(0, CHUNK)],
                    idx_v.at[i % 2], sem_i).wait()

        def issue_gather(s, par):
            @pl.when(valid(s))
            def _():
                for b in range(NBLK):
                    pltpu.async_copy(
                        pos_sp.at[exps[par].at[pl.ds(384 * b, 384)]],
                        valss[par].at[pl.ds(512 * b, 384)], sem_g)

        def wait_gather(s, par):
            @pl.when(valid(s))
            def _():
                pltpu.make_async_copy(
                    pos_sp.at[exps[par]],
                    valss[par].at[pl.ds(0, 3 * CHUNK)], sem_g).wait()

        def issue_out(s, par):
            @pl.when(valid(s))
            def _():
                pltpu.async_copy(
                    valss[par],
                    outs[par].at[pl.ds(4 * chunk_of(s) * CHUNK, 4 * CHUNK)],
                    sem_o)

        def wait_out(s, extra=None):
            pred = valid(s) if extra is None else (valid(s) & extra)

            @pl.when(pred)
            def _():
                pltpu.make_async_copy(
                    vals0_v, out0_hbm.at[pl.ds(0, 4 * CHUNK)], sem_o).wait()

        issue_idx(0)
        wait_idx(0)
        issue_idx(1)
        prepare(0, 0, 0)
        issue_gather(0, 0)

        def body(t, carry):
            for par in (0, 1):
                s = 2 * t + par
                par1 = 1 - par
                if par == 1:
                    wait_idx(t + 1)
                    issue_idx(t + 2)
                    q1 = (t + 1) % 2
                else:
                    q1 = t % 2
                prepare(s + 1, par1, q1)
                wait_out(s - 1, s >= 1)
                issue_gather(s + 1, par1)
                wait_gather(s, par)
                issue_out(s, par)
            return carry

        lax.fori_loop(0, ITERS, body, 0)

        wait_out(NSTEPS - 1)

    return k


_kernel = _make_kernel()


def kernel(position, edge_index):
    out0, out1 = _kernel(position.reshape(-1), edge_index)

    def as2d(flat):
        y = flat.reshape(N_EDGES // 128, 4, 128).transpose(0, 2, 1)
        return y.reshape(N_EDGES, 4)[:, :3]

    return (as2d(out0), as2d(out1))
